# Initial kernel scaffold; baseline (speedup 1.0000x reference)
import jax
import jax.numpy as jnp
from jax.experimental import pallas as pl


def kernel(flat_movie_ids, cu_seqlens, table, W1, b1, W2, b2):
    raise NotImplementedError("write your pallas kernel here")


# speedup vs baseline: 2.2100x; 2.2100x over previous
import jax
import jax.numpy as jnp
from jax import lax
from jax.experimental import pallas as pl
from jax.experimental.pallas import tpu as pltpu
from jax.experimental.pallas import tpu_sc as plsc

_B = 4096
_TOTAL = 204800
_VOCAB = 100000
_D = 128
_H = 512
_HH = _H // 2
_NC = 2
_NS = 16
_K = 128
_UPT = _B // _NS
_CUPAD = 8192
_VR = 1000
_UB = 512


def _mlp1_body(tab_ref, w1_ref, b1_ref, outa_ref, outb_ref):
    h = jnp.dot(tab_ref[...], w1_ref[...], preferred_element_type=jnp.float32)
    h = jnp.maximum(h + b1_ref[...], 0.0)
    outa_ref[...] = h[:, :_HH]
    outb_ref[...] = h[:, _HH:]


def _mlp1(table, w1, b1_2d):
    return pl.pallas_call(
        _mlp1_body,
        grid=(_VOCAB // _VR,),
        in_specs=[
            pl.BlockSpec((_VR, _D), lambda i: (i, 0)),
            pl.BlockSpec((_D, _H), lambda i: (0, 0)),
            pl.BlockSpec((1, _H), lambda i: (0, 0)),
        ],
        out_specs=[
            pl.BlockSpec((_VR, _HH), lambda i: (i, 0)),
            pl.BlockSpec((_VR, _HH), lambda i: (i, 0)),
        ],
        out_shape=[
            jax.ShapeDtypeStruct((_VOCAB, _HH), jnp.float32),
            jax.ShapeDtypeStruct((_VOCAB, _HH), jnp.float32),
        ],
    )(table, w1, b1_2d)


def _scalar(ref, i):
    v = plsc.load_gather(ref, [jnp.full((16,), i, jnp.int32)])
    return jnp.min(v)


def _chunk_loop(s, ids_hbm, cu_v, t1_hbm, ids_v, seg_v, rows_v, acc_v, sem):
    u0 = s * _UPT
    t0 = _scalar(cu_v, u0)
    t1 = _scalar(cu_v, u0 + _UPT)
    t0a = (t0 // 8) * 8
    nchunks = (t1 - t0a + _K - 1) // _K

    def chunk(i, carry):
        base = t0a + i * _K
        pltpu.sync_copy(ids_hbm.at[pl.ds(base, _K)], ids_v)
        gat = pltpu.async_copy(t1_hbm.at[ids_v], rows_v, sem)
        for j in range(_K // 16):
            t = base + j * 16 + lax.iota(jnp.int32, 16)
            pos = jnp.zeros((16,), jnp.int32)
            sv = _CUPAD // 2
            while sv >= 1:
                cand = pos + sv
                vals = plsc.load_gather(cu_v, [cand])
                pos = jnp.where(vals <= t, cand, pos)
                sv //= 2
            seg_v[pl.ds(j * 16, 16)] = pos - u0
        gat.wait()

        def tok(t, cc):
            g = base + t

            @pl.when(jnp.logical_and(g >= t0, g < t1))
            def _():
                lu = _scalar(seg_v, t)
                for c0 in range(_HH // 16):
                    plsc.addupdate(acc_v.at[lu, pl.ds(c0 * 16, 16)],
                                   rows_v[t, pl.ds(c0 * 16, 16)])
            return cc

        lax.fori_loop(0, _K, tok, 0, unroll=False)
        return carry

    lax.fori_loop(0, nchunks, chunk, 0, unroll=False)


def _seg_body(ids_hbm, cu_hbm, t1a_hbm, t1b_hbm, outa_hbm, outb_hbm,
              cu_v, ids_v, seg_v, rows_v, acc_v, sem):
    c = lax.axis_index("c")
    s = lax.axis_index("s")
    row0 = s * _UPT

    pltpu.sync_copy(cu_hbm, cu_v)

    def zrow(r, cc):
        for c0 in range(_HH // 16):
            acc_v[r, pl.ds(c0 * 16, 16)] = jnp.zeros((16,), jnp.float32)
        return cc

    lax.fori_loop(0, _UPT, zrow, 0, unroll=False)

    @pl.when(c == 0)
    def _():
        _chunk_loop(s, ids_hbm, cu_v, t1a_hbm, ids_v, seg_v, rows_v, acc_v,
                    sem)
        pltpu.sync_copy(acc_v, outa_hbm.at[pl.ds(row0, _UPT)])

    @pl.when(c == 1)
    def _():
        _chunk_loop(s, ids_hbm, cu_v, t1b_hbm, ids_v, seg_v, rows_v, acc_v,
                    sem)
        pltpu.sync_copy(acc_v, outb_hbm.at[pl.ds(row0, _UPT)])


def _segsum(ids_padded, cu_pad, t1a, t1b):
    return pl.kernel(
        _seg_body,
        out_type=(
            jax.ShapeDtypeStruct((_B, _HH), jnp.float32),
            jax.ShapeDtypeStruct((_B, _HH), jnp.float32),
        ),
        mesh=plsc.VectorSubcoreMesh(
            core_axis_name="c", subcore_axis_name="s",
            num_cores=_NC, num_subcores=_NS,
        ),
        scratch_types=[
            pltpu.VMEM((_CUPAD,), jnp.int32),
            pltpu.VMEM((_K,), jnp.int32),
            pltpu.VMEM((_K,), jnp.int32),
            pltpu.VMEM((_K, _HH), jnp.float32),
            pltpu.VMEM((_UPT, _HH), jnp.float32),
            pltpu.SemaphoreType.DMA,
        ],
        compiler_params=pltpu.CompilerParams(needs_layout_passes=False),
    )(ids_padded, cu_pad, t1a, t1b)


def _out_body(a_ref, b_ref, lo_ref, hi_ref, w2a_ref, w2b_ref, b2_ref, o_ref):
    cnt = (hi_ref[...] - lo_ref[...]).astype(jnp.float32)
    inv = 1.0 / jnp.maximum(cnt, 1.0)
    y = jnp.dot(a_ref[...] * inv, w2a_ref[...],
                preferred_element_type=jnp.float32)
    y = y + jnp.dot(b_ref[...] * inv, w2b_ref[...],
                    preferred_element_type=jnp.float32)
    y = y + b2_ref[...]
    o_ref[...] = jnp.where(cnt > 0.0, y, jnp.zeros_like(y))


def _finish(suma, sumb, cu_lo, cu_hi, w2a, w2b, b2_2d):
    return pl.pallas_call(
        _out_body,
        grid=(_B // _UB,),
        in_specs=[
            pl.BlockSpec((_UB, _HH), lambda i: (i, 0)),
            pl.BlockSpec((_UB, _HH), lambda i: (i, 0)),
            pl.BlockSpec((_UB, 1), lambda i: (i, 0)),
            pl.BlockSpec((_UB, 1), lambda i: (i, 0)),
            pl.BlockSpec((_HH, _D), lambda i: (0, 0)),
            pl.BlockSpec((_HH, _D), lambda i: (0, 0)),
            pl.BlockSpec((1, _D), lambda i: (0, 0)),
        ],
        out_specs=pl.BlockSpec((_UB, _D), lambda i: (i, 0)),
        out_shape=jax.ShapeDtypeStruct((_B, _D), jnp.float32),
    )(suma, sumb, cu_lo, cu_hi, w2a, w2b, b2_2d)


def kernel(flat_movie_ids, cu_seqlens, table, W1, b1, W2, b2):
    t1a, t1b = _mlp1(table, W1, b1.reshape(1, _H))
    cu_pad = jnp.concatenate([
        cu_seqlens,
        jnp.full((_CUPAD - _B - 1,), jnp.int32(0x3FFFFFFF), jnp.int32),
    ])
    ids_padded = jnp.concatenate([
        flat_movie_ids, jnp.zeros((_K,), jnp.int32),
    ])
    suma, sumb = _segsum(ids_padded, cu_pad, t1a, t1b)
    out = _finish(
        suma, sumb,
        cu_seqlens[:-1].reshape(_B, 1), cu_seqlens[1:].reshape(_B, 1),
        W2[:_HH], W2[_HH:], b2.reshape(1, _D),
    )
    return out


# speedup vs baseline: 5.5056x; 2.4913x over previous
import jax
import jax.numpy as jnp
from jax import lax
from jax.experimental import pallas as pl
from jax.experimental.pallas import tpu as pltpu
from jax.experimental.pallas import tpu_sc as plsc

_B = 4096
_TOTAL = 204800
_VOCAB = 100000
_D = 128
_H = 512
_HH = _H // 2
_NC = 2
_NS = 16
_K = 128
_UPT = _B // _NS
_CUPAD = 8192
_VR = 1000
_UB = 512


def _mlp1_body(tab_ref, w1_ref, b1_ref, outa_ref, outb_ref):
    h = jnp.dot(tab_ref[...], w1_ref[...], preferred_element_type=jnp.float32)
    h = jnp.maximum(h + b1_ref[...], 0.0)
    outa_ref[...] = h[:, :_HH]
    outb_ref[...] = h[:, _HH:]


def _mlp1(table, w1, b1_2d):
    return pl.pallas_call(
        _mlp1_body,
        grid=(_VOCAB // _VR,),
        in_specs=[
            pl.BlockSpec((_VR, _D), lambda i: (i, 0)),
            pl.BlockSpec((_D, _H), lambda i: (0, 0)),
            pl.BlockSpec((1, _H), lambda i: (0, 0)),
        ],
        out_specs=[
            pl.BlockSpec((_VR, _HH), lambda i: (i, 0)),
            pl.BlockSpec((_VR, _HH), lambda i: (i, 0)),
        ],
        out_shape=[
            jax.ShapeDtypeStruct((_VOCAB, _HH), jnp.float32),
            jax.ShapeDtypeStruct((_VOCAB, _HH), jnp.float32),
        ],
    )(table, w1, b1_2d)


def _scal(ref, i):
    return ref[pl.ds(i, 16)][0]


def _chunk_loop(s, ids_hbm, cu_v, t1_hbm, ids_v, rows_v, acc_v, sem):
    u0 = s * _UPT
    t0 = _scal(cu_v, u0)
    t1 = _scal(cu_v, u0 + _UPT)
    t0a = (t0 // 8) * 8
    nchunks = (t1 - t0a + _K - 1) // _K
    nreg = _HH // 16

    def chunk(i, u_in):
        base = t0a + i * _K
        pltpu.sync_copy(ids_hbm.at[pl.ds(base, _K)], ids_v)
        pltpu.async_copy(t1_hbm.at[ids_v], rows_v, sem).wait()
        lo = jnp.maximum(t0, base)
        hi = jnp.minimum(t1, base + _K)

        def ubody(st):
            u, t = st

            def sc(st2):
                return st2[1] <= t

            def sb(st2):
                u2 = st2[0] + 1
                return (u2, _scal(cu_v, u2 + 1))

            u, e_user = lax.while_loop(sc, sb, (u, _scal(cu_v, u + 1)))
            e = jnp.minimum(e_user, hi)
            tl0 = t - base

            def tok(k, regs):
                tl = tl0 + k
                return tuple(regs[c0] + rows_v[tl, pl.ds(c0 * 16, 16)]
                             for c0 in range(nreg))

            regs = lax.fori_loop(
                0, e - t, tok,
                tuple(jnp.zeros((16,), jnp.float32) for _ in range(nreg)),
                unroll=False)
            lu = u - u0
            for c0 in range(nreg):
                plsc.addupdate(acc_v.at[lu, pl.ds(c0 * 16, 16)], regs[c0])
            return (u, e)

        u_out, _ = lax.while_loop(lambda st: st[1] < hi, ubody, (u_in, lo))
        return u_out

    lax.fori_loop(0, nchunks, chunk, u0, unroll=False)


def _seg_body(ids_hbm, cu_hbm, t1a_hbm, t1b_hbm, outa_hbm, outb_hbm,
              cu_v, ids_v, rows_v, acc_v, sem):
    c = lax.axis_index("c")
    s = lax.axis_index("s")
    row0 = s * _UPT

    pltpu.sync_copy(cu_hbm, cu_v)

    def zrow(r, cc):
        for c0 in range(_HH // 16):
            acc_v[r, pl.ds(c0 * 16, 16)] = jnp.zeros((16,), jnp.float32)
        return cc

    lax.fori_loop(0, _UPT, zrow, 0, unroll=False)

    @pl.when(c == 0)
    def _():
        _chunk_loop(s, ids_hbm, cu_v, t1a_hbm, ids_v, rows_v, acc_v, sem)
        pltpu.sync_copy(acc_v, outa_hbm.at[pl.ds(row0, _UPT)])

    @pl.when(c == 1)
    def _():
        _chunk_loop(s, ids_hbm, cu_v, t1b_hbm, ids_v, rows_v, acc_v, sem)
        pltpu.sync_copy(acc_v, outb_hbm.at[pl.ds(row0, _UPT)])


def _segsum(ids_padded, cu_pad, t1a, t1b):
    return pl.kernel(
        _seg_body,
        out_type=(
            jax.ShapeDtypeStruct((_B, _HH), jnp.float32),
            jax.ShapeDtypeStruct((_B, _HH), jnp.float32),
        ),
        mesh=plsc.VectorSubcoreMesh(
            core_axis_name="c", subcore_axis_name="s",
            num_cores=_NC, num_subcores=_NS,
        ),
        scratch_types=[
            pltpu.VMEM((_CUPAD,), jnp.int32),
            pltpu.VMEM((_K,), jnp.int32),
            pltpu.VMEM((_K, _HH), jnp.float32),
            pltpu.VMEM((_UPT, _HH), jnp.float32),
            pltpu.SemaphoreType.DMA,
        ],
        compiler_params=pltpu.CompilerParams(needs_layout_passes=False),
    )(ids_padded, cu_pad, t1a, t1b)


def _out_body(a_ref, b_ref, lo_ref, hi_ref, w2a_ref, w2b_ref, b2_ref, o_ref):
    cnt = (hi_ref[...] - lo_ref[...]).astype(jnp.float32)
    inv = 1.0 / jnp.maximum(cnt, 1.0)
    y = jnp.dot(a_ref[...] * inv, w2a_ref[...],
                preferred_element_type=jnp.float32)
    y = y + jnp.dot(b_ref[...] * inv, w2b_ref[...],
                    preferred_element_type=jnp.float32)
    y = y + b2_ref[...]
    o_ref[...] = jnp.where(cnt > 0.0, y, jnp.zeros_like(y))


def _finish(suma, sumb, cu_lo, cu_hi, w2a, w2b, b2_2d):
    return pl.pallas_call(
        _out_body,
        grid=(_B // _UB,),
        in_specs=[
            pl.BlockSpec((_UB, _HH), lambda i: (i, 0)),
            pl.BlockSpec((_UB, _HH), lambda i: (i, 0)),
            pl.BlockSpec((_UB, 1), lambda i: (i, 0)),
            pl.BlockSpec((_UB, 1), lambda i: (i, 0)),
            pl.BlockSpec((_HH, _D), lambda i: (0, 0)),
            pl.BlockSpec((_HH, _D), lambda i: (0, 0)),
            pl.BlockSpec((1, _D), lambda i: (0, 0)),
        ],
        out_specs=pl.BlockSpec((_UB, _D), lambda i: (i, 0)),
        out_shape=jax.ShapeDtypeStruct((_B, _D), jnp.float32),
    )(suma, sumb, cu_lo, cu_hi, w2a, w2b, b2_2d)


def kernel(flat_movie_ids, cu_seqlens, table, W1, b1, W2, b2):
    t1a, t1b = _mlp1(table, W1, b1.reshape(1, _H))
    cu_pad = jnp.concatenate([
        cu_seqlens,
        jnp.full((_CUPAD - _B - 1,), jnp.int32(0x3FFFFFFF), jnp.int32),
    ])
    ids_padded = jnp.concatenate([
        flat_movie_ids, jnp.zeros((_K,), jnp.int32),
    ])
    suma, sumb = _segsum(ids_padded, cu_pad, t1a, t1b)
    out = _finish(
        suma, sumb,
        cu_seqlens[:-1].reshape(_B, 1), cu_seqlens[1:].reshape(_B, 1),
        W2[:_HH], W2[_HH:], b2.reshape(1, _D),
    )
    return out


# speedup vs baseline: 7.7133x; 1.4010x over previous
import jax
import jax.numpy as jnp
from jax import lax
from jax.experimental import pallas as pl
from jax.experimental.pallas import tpu as pltpu
from jax.experimental.pallas import tpu_sc as plsc

_B = 4096
_TOTAL = 204800
_VOCAB = 100000
_D = 128
_H = 512
_HH = _H // 2
_NC = 2
_NS = 16
_K = 96
_UPT = _B // _NS
_CUPAD = 4224
_VR = 1000
_UB = 512


def _mlp1_body(tab_ref, w1_ref, b1_ref, outa_ref, outb_ref):
    h = jnp.dot(tab_ref[...], w1_ref[...], preferred_element_type=jnp.float32)
    h = jnp.maximum(h + b1_ref[...], 0.0)
    outa_ref[...] = h[:, :_HH]
    outb_ref[...] = h[:, _HH:]


def _mlp1(table, w1, b1_2d):
    return pl.pallas_call(
        _mlp1_body,
        grid=(_VOCAB // _VR,),
        in_specs=[
            pl.BlockSpec((_VR, _D), lambda i: (i, 0)),
            pl.BlockSpec((_D, _H), lambda i: (0, 0)),
            pl.BlockSpec((1, _H), lambda i: (0, 0)),
        ],
        out_specs=[
            pl.BlockSpec((_VR, _HH), lambda i: (i, 0)),
            pl.BlockSpec((_VR, _HH), lambda i: (i, 0)),
        ],
        out_shape=[
            jax.ShapeDtypeStruct((_VOCAB, _HH), jnp.float32),
            jax.ShapeDtypeStruct((_VOCAB, _HH), jnp.float32),
        ],
    )(table, w1, b1_2d)


def _scal(ref, i):
    return ref[pl.ds(i, 16)][0]


def _chunk_loop(s, ids_hbm, cu_v, t1_hbm, ids0, ids1, rows0, rows1, acc_v,
                sem0, sem1):
    u0 = s * _UPT
    t0 = _scal(cu_v, u0)
    t1 = _scal(cu_v, u0 + _UPT)
    t0a = (t0 // 8) * 8
    nchunks = (t1 - t0a + _K - 1) // _K
    npairs = (nchunks + 1) // 2
    nreg = _HH // 16

    def issue(base, ids_b, rows_b, sem_b):
        pltpu.sync_copy(ids_hbm.at[pl.ds(base, _K)], ids_b)
        pltpu.async_copy(t1_hbm.at[ids_b], rows_b, sem_b)

    def walk(u_in, base, rows_b):
        lo = jnp.maximum(t0, base)
        hi = jnp.minimum(t1, base + _K)

        def ubody(st):
            u, t = st

            def sc(st2):
                return st2[1] <= t

            def sb(st2):
                u2 = st2[0] + 1
                return (u2, _scal(cu_v, u2 + 1))

            u, e_user = lax.while_loop(sc, sb, (u, _scal(cu_v, u + 1)))
            e = jnp.minimum(e_user, hi)
            tl0 = t - base

            def tok(k, regs):
                tl = tl0 + k
                return tuple(regs[c0] + rows_b[tl, pl.ds(c0 * 16, 16)]
                             for c0 in range(nreg))

            regs = lax.fori_loop(
                0, e - t, tok,
                tuple(jnp.zeros((16,), jnp.float32) for _ in range(nreg)),
                unroll=False)
            lu = u - u0
            for c0 in range(nreg):
                plsc.addupdate(acc_v.at[lu, pl.ds(c0 * 16, 16)], regs[c0])
            return (u, e)

        u_out, _ = lax.while_loop(lambda st: st[1] < hi, ubody, (u_in, lo))
        return u_out

    issue(t0a, ids0, rows0, sem0)

    def pair(p, u):
        b0 = t0a + (2 * p) * _K
        issue(b0 + _K, ids1, rows1, sem1)
        pltpu.make_async_copy(t1_hbm.at[ids0], rows0, sem0).wait()
        u = walk(u, b0, rows0)
        issue(b0 + 2 * _K, ids0, rows0, sem0)
        pltpu.make_async_copy(t1_hbm.at[ids1], rows1, sem1).wait()
        u = walk(u, b0 + _K, rows1)
        return u

    u_fin = lax.fori_loop(0, npairs, pair, u0, unroll=False)
    pltpu.make_async_copy(t1_hbm.at[ids0], rows0, sem0).wait()
    return u_fin


def _seg_body(ids_hbm, cu_hbm, t1a_hbm, t1b_hbm, outa_hbm, outb_hbm,
              cu_v, ids0, ids1, rows0, rows1, acc_v, sem0, sem1):
    c = lax.axis_index("c")
    s = lax.axis_index("s")
    row0 = s * _UPT

    pltpu.sync_copy(cu_hbm, cu_v)

    def zrow(r, cc):
        for c0 in range(_HH // 16):
            acc_v[r, pl.ds(c0 * 16, 16)] = jnp.zeros((16,), jnp.float32)
        return cc

    lax.fori_loop(0, _UPT, zrow, 0, unroll=False)

    @pl.when(c == 0)
    def _():
        _chunk_loop(s, ids_hbm, cu_v, t1a_hbm, ids0, ids1, rows0, rows1,
                    acc_v, sem0, sem1)
        pltpu.sync_copy(acc_v, outa_hbm.at[pl.ds(row0, _UPT)])

    @pl.when(c == 1)
    def _():
        _chunk_loop(s, ids_hbm, cu_v, t1b_hbm, ids0, ids1, rows0, rows1,
                    acc_v, sem0, sem1)
        pltpu.sync_copy(acc_v, outb_hbm.at[pl.ds(row0, _UPT)])


def _segsum(ids_padded, cu_pad, t1a, t1b):
    return pl.kernel(
        _seg_body,
        out_type=(
            jax.ShapeDtypeStruct((_B, _HH), jnp.float32),
            jax.ShapeDtypeStruct((_B, _HH), jnp.float32),
        ),
        mesh=plsc.VectorSubcoreMesh(
            core_axis_name="c", subcore_axis_name="s",
            num_cores=_NC, num_subcores=_NS,
        ),
        scratch_types=[
            pltpu.VMEM((_CUPAD,), jnp.int32),
            pltpu.VMEM((_K,), jnp.int32),
            pltpu.VMEM((_K,), jnp.int32),
            pltpu.VMEM((_K, _HH), jnp.float32),
            pltpu.VMEM((_K, _HH), jnp.float32),
            pltpu.VMEM((_UPT, _HH), jnp.float32),
            pltpu.SemaphoreType.DMA,
            pltpu.SemaphoreType.DMA,
        ],
        compiler_params=pltpu.CompilerParams(needs_layout_passes=False),
    )(ids_padded, cu_pad, t1a, t1b)


def _out_body(a_ref, b_ref, lo_ref, hi_ref, w2a_ref, w2b_ref, b2_ref, o_ref):
    cnt = (hi_ref[...] - lo_ref[...]).astype(jnp.float32)
    inv = 1.0 / jnp.maximum(cnt, 1.0)
    y = jnp.dot(a_ref[...] * inv, w2a_ref[...],
                preferred_element_type=jnp.float32)
    y = y + jnp.dot(b_ref[...] * inv, w2b_ref[...],
                    preferred_element_type=jnp.float32)
    y = y + b2_ref[...]
    o_ref[...] = jnp.where(cnt > 0.0, y, jnp.zeros_like(y))


def _finish(suma, sumb, cu_lo, cu_hi, w2a, w2b, b2_2d):
    return pl.pallas_call(
        _out_body,
        grid=(_B // _UB,),
        in_specs=[
            pl.BlockSpec((_UB, _HH), lambda i: (i, 0)),
            pl.BlockSpec((_UB, _HH), lambda i: (i, 0)),
            pl.BlockSpec((_UB, 1), lambda i: (i, 0)),
            pl.BlockSpec((_UB, 1), lambda i: (i, 0)),
            pl.BlockSpec((_HH, _D), lambda i: (0, 0)),
            pl.BlockSpec((_HH, _D), lambda i: (0, 0)),
            pl.BlockSpec((1, _D), lambda i: (0, 0)),
        ],
        out_specs=pl.BlockSpec((_UB, _D), lambda i: (i, 0)),
        out_shape=jax.ShapeDtypeStruct((_B, _D), jnp.float32),
    )(suma, sumb, cu_lo, cu_hi, w2a, w2b, b2_2d)


def kernel(flat_movie_ids, cu_seqlens, table, W1, b1, W2, b2):
    t1a, t1b = _mlp1(table, W1, b1.reshape(1, _H))
    cu_pad = jnp.concatenate([
        cu_seqlens,
        jnp.full((_CUPAD - _B - 1,), jnp.int32(0x3FFFFFFF), jnp.int32),
    ])
    ids_padded = jnp.concatenate([
        flat_movie_ids, jnp.zeros((3 * _K,), jnp.int32),
    ])
    suma, sumb = _segsum(ids_padded, cu_pad, t1a, t1b)
    out = _finish(
        suma, sumb,
        cu_seqlens[:-1].reshape(_B, 1), cu_seqlens[1:].reshape(_B, 1),
        W2[:_HH], W2[_HH:], b2.reshape(1, _D),
    )
    return out


# speedup vs baseline: 9.7886x; 1.2691x over previous
import jax
import jax.numpy as jnp
import numpy as np
from jax import lax
from jax.experimental import pallas as pl
from jax.experimental.pallas import tpu as pltpu
from jax.experimental.pallas import tpu_sc as plsc

_B = 4096
_TOTAL = 204800
_VOCAB = 100000
_D = 128
_H = 512
_HH = _H // 2
_NC = 2
_NS = 16
_K = 128
_HW = _HH // 2
_UPT = _B // _NS
_CUPAD = 4224
_VR = 1000
_UB = 512


def _pack_bf16_pair(x):
    def rnd(v):
        b = lax.bitcast_convert_type(v, jnp.int32)
        return b + jnp.int32(0x7FFF) + lax.bitwise_and(
            lax.shift_right_logical(b, 16), jnp.int32(1))

    w = lax.bitwise_or(
        lax.shift_right_logical(rnd(x[:, :_HW]), 16),
        lax.bitwise_and(rnd(x[:, _HW:]), jnp.int32(-65536)))
    return lax.bitcast_convert_type(w, jnp.float32)


def _mlp1_body(tab_ref, w1_ref, b1_ref, outa_ref, outb_ref):
    h = jnp.dot(tab_ref[...], w1_ref[...], preferred_element_type=jnp.float32)
    h = jnp.maximum(h + b1_ref[...], 0.0)
    outa_ref[...] = _pack_bf16_pair(h[:, :_HH])
    outb_ref[...] = _pack_bf16_pair(h[:, _HH:])


def _mlp1(table, w1, b1_2d):
    return pl.pallas_call(
        _mlp1_body,
        grid=(_VOCAB // _VR,),
        in_specs=[
            pl.BlockSpec((_VR, _D), lambda i: (i, 0)),
            pl.BlockSpec((_D, _H), lambda i: (0, 0)),
            pl.BlockSpec((1, _H), lambda i: (0, 0)),
        ],
        out_specs=[
            pl.BlockSpec((_VR, _HW), lambda i: (i, 0)),
            pl.BlockSpec((_VR, _HW), lambda i: (i, 0)),
        ],
        out_shape=[
            jax.ShapeDtypeStruct((_VOCAB, _HW), jnp.float32),
            jax.ShapeDtypeStruct((_VOCAB, _HW), jnp.float32),
        ],
    )(table, w1, b1_2d)


def _scal(ref, i):
    return ref[pl.ds(i, 16)][0]


def _chunk_loop(s, ids_hbm, cu_v, t1_hbm, ids0, ids1, rows0, rows1, acc_v,
                sem0, sem1):
    u0 = s * _UPT
    t0 = _scal(cu_v, u0)
    t1 = _scal(cu_v, u0 + _UPT)
    t0a = (t0 // 8) * 8
    nchunks = (t1 - t0a + _K - 1) // _K
    npairs = (nchunks + 1) // 2
    nreg = _HH // 16

    def issue(base, ids_b, rows_b, sem_b):
        pltpu.sync_copy(ids_hbm.at[pl.ds(base, _K)], ids_b)
        pltpu.async_copy(t1_hbm.at[ids_b], rows_b, sem_b)

    def walk(u_in, base, rows_b):
        lo = jnp.maximum(t0, base)
        hi = jnp.minimum(t1, base + _K)

        def ubody(st):
            u, t = st

            def sc(st2):
                return st2[1] <= t

            def sb(st2):
                u2 = st2[0] + 1
                return (u2, _scal(cu_v, u2 + 1))

            u, e_user = lax.while_loop(sc, sb, (u, _scal(cu_v, u + 1)))
            e = jnp.minimum(e_user, hi)
            tl0 = t - base

            def tok(k, regs):
                tl = tl0 + k
                new = list(regs)
                for g in range(8):
                    w = plsc.bitcast(rows_b[tl, pl.ds(g * 16, 16)],
                                     jnp.int32)
                    lo16 = plsc.bitcast(lax.shift_left(w, 16), jnp.float32)
                    hi16 = plsc.bitcast(
                        lax.bitwise_and(w, jnp.int32(-65536)), jnp.float32)
                    new[g] = regs[g] + lo16
                    new[8 + g] = regs[8 + g] + hi16
                return tuple(new)

            regs = lax.fori_loop(
                0, e - t, tok,
                tuple(jnp.zeros((16,), jnp.float32) for _ in range(nreg)),
                unroll=False)
            lu = u - u0
            for c0 in range(nreg):
                plsc.addupdate(acc_v.at[lu, pl.ds(c0 * 16, 16)], regs[c0])
            return (u, e)

        u_out, _ = lax.while_loop(lambda st: st[1] < hi, ubody, (u_in, lo))
        return u_out

    issue(t0a, ids0, rows0, sem0)

    def pair(p, u):
        b0 = t0a + (2 * p) * _K
        issue(b0 + _K, ids1, rows1, sem1)
        pltpu.make_async_copy(t1_hbm.at[ids0], rows0, sem0).wait()
        u = walk(u, b0, rows0)
        issue(b0 + 2 * _K, ids0, rows0, sem0)
        pltpu.make_async_copy(t1_hbm.at[ids1], rows1, sem1).wait()
        u = walk(u, b0 + _K, rows1)
        return u

    u_fin = lax.fori_loop(0, npairs, pair, u0, unroll=False)
    pltpu.make_async_copy(t1_hbm.at[ids0], rows0, sem0).wait()
    return u_fin


def _seg_body(ids_hbm, cu_hbm, t1a_hbm, t1b_hbm, outa_hbm, outb_hbm,
              cu_v, ids0, ids1, rows0, rows1, acc_v, sem0, sem1):
    c = lax.axis_index("c")
    s = lax.axis_index("s")
    row0 = s * _UPT

    pltpu.sync_copy(cu_hbm, cu_v)

    def zrow(r, cc):
        for c0 in range(_HH // 16):
            acc_v[r, pl.ds(c0 * 16, 16)] = jnp.zeros((16,), jnp.float32)
        return cc

    lax.fori_loop(0, _UPT, zrow, 0, unroll=False)

    @pl.when(c == 0)
    def _():
        _chunk_loop(s, ids_hbm, cu_v, t1a_hbm, ids0, ids1, rows0, rows1,
                    acc_v, sem0, sem1)
        pltpu.sync_copy(acc_v, outa_hbm.at[pl.ds(row0, _UPT)])

    @pl.when(c == 1)
    def _():
        _chunk_loop(s, ids_hbm, cu_v, t1b_hbm, ids0, ids1, rows0, rows1,
                    acc_v, sem0, sem1)
        pltpu.sync_copy(acc_v, outb_hbm.at[pl.ds(row0, _UPT)])


def _segsum(ids_padded, cu_pad, t1a, t1b):
    return pl.kernel(
        _seg_body,
        out_type=(
            jax.ShapeDtypeStruct((_B, _HH), jnp.float32),
            jax.ShapeDtypeStruct((_B, _HH), jnp.float32),
        ),
        mesh=plsc.VectorSubcoreMesh(
            core_axis_name="c", subcore_axis_name="s",
            num_cores=_NC, num_subcores=_NS,
        ),
        scratch_types=[
            pltpu.VMEM((_CUPAD,), jnp.int32),
            pltpu.VMEM((_K,), jnp.int32),
            pltpu.VMEM((_K,), jnp.int32),
            pltpu.VMEM((_K, _HW), jnp.float32),
            pltpu.VMEM((_K, _HW), jnp.float32),
            pltpu.VMEM((_UPT, _HH), jnp.float32),
            pltpu.SemaphoreType.DMA,
            pltpu.SemaphoreType.DMA,
        ],
        compiler_params=pltpu.CompilerParams(needs_layout_passes=False),
    )(ids_padded, cu_pad, t1a, t1b)


def _out_body(a_ref, b_ref, lo_ref, hi_ref, w2a_ref, w2b_ref, b2_ref, o_ref):
    cnt = (hi_ref[...] - lo_ref[...]).astype(jnp.float32)
    inv = 1.0 / jnp.maximum(cnt, 1.0)
    y = jnp.dot(a_ref[...] * inv, w2a_ref[...],
                preferred_element_type=jnp.float32)
    y = y + jnp.dot(b_ref[...] * inv, w2b_ref[...],
                    preferred_element_type=jnp.float32)
    y = y + b2_ref[...]
    o_ref[...] = jnp.where(cnt > 0.0, y, jnp.zeros_like(y))


def _finish(suma, sumb, cu_lo, cu_hi, w2a, w2b, b2_2d):
    return pl.pallas_call(
        _out_body,
        grid=(_B // _UB,),
        in_specs=[
            pl.BlockSpec((_UB, _HH), lambda i: (i, 0)),
            pl.BlockSpec((_UB, _HH), lambda i: (i, 0)),
            pl.BlockSpec((_UB, 1), lambda i: (i, 0)),
            pl.BlockSpec((_UB, 1), lambda i: (i, 0)),
            pl.BlockSpec((_HH, _D), lambda i: (0, 0)),
            pl.BlockSpec((_HH, _D), lambda i: (0, 0)),
            pl.BlockSpec((1, _D), lambda i: (0, 0)),
        ],
        out_specs=pl.BlockSpec((_UB, _D), lambda i: (i, 0)),
        out_shape=jax.ShapeDtypeStruct((_B, _D), jnp.float32),
    )(suma, sumb, cu_lo, cu_hi, w2a, w2b, b2_2d)


def kernel(flat_movie_ids, cu_seqlens, table, W1, b1, W2, b2):
    t1a, t1b = _mlp1(table, W1, b1.reshape(1, _H))
    cu_pad = jnp.concatenate([
        cu_seqlens,
        jnp.full((_CUPAD - _B - 1,), jnp.int32(0x3FFFFFFF), jnp.int32),
    ])
    ids_padded = jnp.concatenate([
        flat_movie_ids, jnp.zeros((3 * _K,), jnp.int32),
    ])
    suma, sumb = _segsum(ids_padded, cu_pad, t1a, t1b)
    out = _finish(
        suma, sumb,
        cu_seqlens[:-1].reshape(_B, 1), cu_seqlens[1:].reshape(_B, 1),
        W2[:_HH], W2[_HH:], b2.reshape(1, _D),
    )
    return out
